# baseline (speedup 1.0000x reference)
import functools

import jax
import jax.numpy as jnp
from jax import lax
from jax.experimental import pallas as pl
from jax.experimental.pallas import tpu as pltpu
from jax.experimental.pallas import tpu_sc as plsc

D = 64
NW = 32
CH = 128


@functools.partial(jax.jit, static_argnames=())
def _lookup(idx3d, table):
    n_ch = idx3d.shape[1]
    B = NW * n_ch * CH

    mesh = plsc.VectorSubcoreMesh(core_axis_name="c", subcore_axis_name="s")

    @functools.partial(
        pl.kernel,
        out_type=jax.ShapeDtypeStruct((B, D), jnp.float32),
        mesh=mesh,
        scratch_types=[
            pltpu.VMEM((n_ch, CH), jnp.int32),
            pltpu.VMEM((CH, D), jnp.float32),
            pltpu.SemaphoreType.DMA,
        ],
        compiler_params=pltpu.CompilerParams(use_tc_tiling_on_sc=False),
    )
    def k(idx_hbm, table_hbm, out_hbm, idx_v, rows_v, sem):
        wid = lax.axis_index("s") * 2 + lax.axis_index("c")
        pltpu.sync_copy(idx_hbm.at[wid], idx_v)

        def body(t, carry):
            pltpu.async_copy(table_hbm.at[idx_v.at[t]], rows_v, sem).wait()
            pltpu.sync_copy(rows_v, out_hbm.at[pl.ds((wid * n_ch + t) * CH, CH)])
            return carry

        lax.fori_loop(0, n_ch, body, 0)

    return k(idx3d, table)


def kernel(inputs, emb_edges):
    B = inputs.shape[0] * inputs.shape[1]
    idx3d = inputs.reshape(NW, B // (NW * CH), CH)
    out = _lookup(idx3d, emb_edges)
    return out.reshape(inputs.shape[0], inputs.shape[1], D)


# speedup vs baseline: 1.0264x; 1.0264x over previous
import functools

import jax
import jax.numpy as jnp
from jax import lax
from jax.experimental import pallas as pl
from jax.experimental.pallas import tpu as pltpu
from jax.experimental.pallas import tpu_sc as plsc

D = 64
NW = 32
CH = 256


@functools.partial(jax.jit, static_argnames=())
def _lookup(idx3d, table):
    n_ch = idx3d.shape[1]
    B = NW * n_ch * CH

    mesh = plsc.VectorSubcoreMesh(core_axis_name="c", subcore_axis_name="s")

    @functools.partial(
        pl.kernel,
        out_type=jax.ShapeDtypeStruct((B, D), jnp.float32),
        mesh=mesh,
        scratch_types=[
            pltpu.VMEM((n_ch, CH), jnp.int32),
            pltpu.VMEM((CH, D), jnp.float32),
            pltpu.SemaphoreType.DMA,
        ],
        compiler_params=pltpu.CompilerParams(use_tc_tiling_on_sc=False),
    )
    def k(idx_hbm, table_hbm, out_hbm, idx_v, rows_v, sem):
        wid = lax.axis_index("s") * 2 + lax.axis_index("c")
        pltpu.sync_copy(idx_hbm.at[wid], idx_v)

        def body(t, carry):
            pltpu.async_copy(table_hbm.at[idx_v.at[t]], rows_v, sem).wait()
            pltpu.sync_copy(rows_v, out_hbm.at[pl.ds((wid * n_ch + t) * CH, CH)])
            return carry

        lax.fori_loop(0, n_ch, body, 0)

    return k(idx3d, table)


def kernel(inputs, emb_edges):
    B = inputs.shape[0] * inputs.shape[1]
    idx3d = inputs.reshape(NW, B // (NW * CH), CH)
    out = _lookup(idx3d, emb_edges)
    return out.reshape(inputs.shape[0], inputs.shape[1], D)


# speedup vs baseline: 1.0505x; 1.0235x over previous
import functools

import jax
import jax.numpy as jnp
from jax import lax
from jax.experimental import pallas as pl
from jax.experimental.pallas import tpu as pltpu
from jax.experimental.pallas import tpu_sc as plsc

D = 64
NW = 32
CH = 256
NBUF = 5


@jax.jit
def _lookup(idx3d, table):
    n_ch = idx3d.shape[1]
    B = NW * n_ch * CH
    n_grp = n_ch // NBUF
    assert n_ch % NBUF == 0

    mesh = plsc.VectorSubcoreMesh(core_axis_name="c", subcore_axis_name="s")

    @functools.partial(
        pl.kernel,
        out_type=jax.ShapeDtypeStruct((B, D), jnp.float32),
        mesh=mesh,
        scratch_types=[
            pltpu.VMEM((n_ch, CH), jnp.int32),
            pltpu.VMEM((NBUF, CH, D), jnp.float32),
            pltpu.SemaphoreType.DMA((NBUF,)),
            pltpu.SemaphoreType.DMA((NBUF,)),
        ],
        compiler_params=pltpu.CompilerParams(use_tc_tiling_on_sc=False),
    )
    def k(idx_hbm, table_hbm, out_hbm, idx_v, rows_v, gsem, ssem):
        wid = lax.axis_index("s") * 2 + lax.axis_index("c")
        base = wid * n_ch * CH
        pltpu.sync_copy(idx_hbm.at[wid], idx_v)

        def gather(t, b):
            return pltpu.make_async_copy(
                table_hbm.at[idx_v.at[t]], rows_v.at[b], gsem.at[b])

        def store(t, b):
            return pltpu.make_async_copy(
                rows_v.at[b], out_hbm.at[pl.ds(base + t * CH, CH)], ssem.at[b])

        for b in range(NBUF):
            gather(b, b).start()

        def group(g, carry):
            for b in range(NBUF):
                t = g * NBUF + b
                gather(t, b).wait()
                store(t, b).start()

                @pl.when(g + 1 < n_grp)
                def _():
                    store(t, b).wait()
                    gather(t + NBUF, b).start()
            return carry

        lax.fori_loop(0, n_grp, group, 0)

        for b in range(NBUF):
            t = (n_grp - 1) * NBUF + b
            store(t, b).wait()

    return k(idx3d, table)


def kernel(inputs, emb_edges):
    B = inputs.shape[0] * inputs.shape[1]
    idx3d = inputs.reshape(NW, B // (NW * CH), CH)
    out = _lookup(idx3d, emb_edges)
    return out.reshape(inputs.shape[0], inputs.shape[1], D)
